# Initial kernel scaffold; baseline (speedup 1.0000x reference)
import jax
import jax.numpy as jnp
from jax.experimental import pallas as pl


def kernel(ids, feats, adj, layer_idx, table, W, b):
    raise NotImplementedError("write your pallas kernel here")


# speedup vs baseline: 1.3828x; 1.3828x over previous
import functools

import jax
import jax.numpy as jnp
from jax import lax
from jax.experimental import pallas as pl
from jax.experimental.pallas import tpu as pltpu
from jax.experimental.pallas import tpu_sc as plsc

_N_NODES = 1000000
_EMB = 32
_FEAT = 128
_OUT = _FEAT + _EMB

_NC = 2
_NS = 16
_NW = _NC * _NS
_B_PAD = 102400


def _sc_gather(table, idx):
    b_per_w = _B_PAD // _NW
    mesh = plsc.VectorSubcoreMesh(core_axis_name="c", subcore_axis_name="s")

    @functools.partial(
        pl.kernel,
        mesh=mesh,
        out_type=jax.ShapeDtypeStruct((_B_PAD, _EMB), jnp.float32),
        compiler_params=pltpu.CompilerParams(use_tc_tiling_on_sc=False),
        scratch_types=[
            pltpu.VMEM((b_per_w,), jnp.int32),
            pltpu.VMEM((b_per_w, _EMB), jnp.float32),
            pltpu.SemaphoreType.DMA,
        ],
    )
    def k(idx_hbm, table_hbm, out_hbm, idx_v, rows_v, sem):
        wid = lax.axis_index("s") * _NC + lax.axis_index("c")
        base = wid * b_per_w
        pltpu.sync_copy(idx_hbm.at[pl.ds(base, b_per_w)], idx_v)
        pltpu.async_copy(table_hbm.at[idx_v], rows_v, sem).wait()
        pltpu.sync_copy(rows_v, out_hbm.at[pl.ds(base, b_per_w)])

    return k(idx, table)


def _tc_body(feats_ref, embs_ref, wt_ref, b_ref, out_ref):
    out_ref[:, :_FEAT] = feats_ref[...]
    out_ref[:, _FEAT:] = (
        jnp.dot(embs_ref[...], wt_ref[...], preferred_element_type=jnp.float32)
        + b_ref[...]
    )


def _tc_concat_proj(feats, embs_pad, wt, b2):
    batch = feats.shape[0]
    rows = 2000
    grid = batch // rows
    return pl.pallas_call(
        _tc_body,
        grid=(grid,),
        in_specs=[
            pl.BlockSpec((rows, _FEAT), lambda i: (i, 0)),
            pl.BlockSpec((rows, _EMB), lambda i: (i, 0)),
            pl.BlockSpec((_EMB, _EMB), lambda i: (0, 0)),
            pl.BlockSpec((1, _EMB), lambda i: (0, 0)),
        ],
        out_specs=pl.BlockSpec((rows, _OUT), lambda i: (i, 0)),
        out_shape=jax.ShapeDtypeStruct((batch, _OUT), jnp.float32),
    )(feats, embs_pad, wt, b2)


def kernel(ids, feats, adj, layer_idx, table, W, b):
    batch = ids.shape[0]
    gidx = jnp.where(layer_idx > 0, ids.astype(jnp.int32), jnp.int32(_N_NODES))
    gidx = jnp.concatenate(
        [gidx, jnp.zeros((_B_PAD - batch,), jnp.int32)]
    )
    embs = _sc_gather(table, gidx)
    return _tc_concat_proj(feats, embs, W.T, b.reshape(1, _EMB))


# speedup vs baseline: 1.5136x; 1.0946x over previous
import functools

import jax
import jax.numpy as jnp
from jax import lax
from jax.experimental import pallas as pl
from jax.experimental.pallas import tpu as pltpu
from jax.experimental.pallas import tpu_sc as plsc

_N_NODES = 1000000
_EMB = 32
_FEAT = 128
_OUT = _FEAT + _EMB
_BATCH = 100000

_NC = 2
_NS = 16
_NW = _NC * _NS
_B_PAD = 102400
_BPW = _B_PAD // _NW
_CHUNK = 800
_NCHUNK = _BPW // _CHUNK


def _sc_gather_t(table, idx):
    mesh = plsc.VectorSubcoreMesh(core_axis_name="c", subcore_axis_name="s")

    @functools.partial(
        pl.kernel,
        mesh=mesh,
        out_type=jax.ShapeDtypeStruct((_EMB, _B_PAD), jnp.float32),
        compiler_params=pltpu.CompilerParams(
            use_tc_tiling_on_sc=False, needs_layout_passes=False
        ),
        scratch_types=[
            pltpu.VMEM((_NCHUNK, _CHUNK), jnp.int32),
            pltpu.VMEM((_CHUNK, _EMB), jnp.float32),
            pltpu.VMEM((_EMB, _CHUNK), jnp.float32),
            pltpu.SemaphoreType.DMA,
        ],
    )
    def k(idx_hbm, table_hbm, out_hbm, idx_v, rows_v, tr_v, sem):
        wid = lax.axis_index("s") * _NC + lax.axis_index("c")
        base = wid * _BPW
        lanes = lax.iota(jnp.int32, 16)
        for h in range(_NCHUNK):
            pltpu.sync_copy(
                idx_hbm.at[pl.ds(base + h * _CHUNK, _CHUNK)], idx_v.at[h]
            )
            pltpu.async_copy(table_hbm.at[idx_v.at[h]], rows_v, sem).wait()

            def body(kb, carry):
                kidx = kb * 16 + lanes
                for j in range(_EMB):
                    vals = plsc.load_gather(
                        rows_v, [kidx, jnp.full((16,), j, jnp.int32)]
                    )
                    tr_v.at[j][pl.ds(kb * 16, 16)] = vals
                return carry

            lax.fori_loop(0, _CHUNK // 16, body, 0, unroll=False)
            pltpu.sync_copy(
                tr_v, out_hbm.at[:, pl.ds(base + h * _CHUNK, _CHUNK)]
            )

    return k(idx, table)


def _tc_feats_body(feats_ref, out_ref):
    out_ref[...] = feats_ref[...].T


def _tc_emb_body(embst_ref, w_ref, b_ref, _outp_ref, out_ref):
    out_ref[...] = (
        jnp.dot(w_ref[...], embst_ref[...], preferred_element_type=jnp.float32)
        + b_ref[...]
    )


_COLS = 2048


def _tc_concat_proj(feats, embst, w, b2):
    grid = pl.cdiv(_BATCH, _COLS)
    outt0 = pl.pallas_call(
        _tc_feats_body,
        grid=(grid,),
        in_specs=[pl.BlockSpec((_COLS, _FEAT), lambda i: (i, 0))],
        out_specs=pl.BlockSpec((_FEAT, _COLS), lambda i: (0, i)),
        out_shape=jax.ShapeDtypeStruct((_OUT, _BATCH), jnp.float32),
    )(feats)
    outt = pl.pallas_call(
        _tc_emb_body,
        grid=(grid,),
        in_specs=[
            pl.BlockSpec((_EMB, _COLS), lambda i: (0, i)),
            pl.BlockSpec((_EMB, _EMB), lambda i: (0, 0)),
            pl.BlockSpec((_EMB, 1), lambda i: (0, 0)),
            pl.BlockSpec(memory_space=pl.ANY),
        ],
        out_specs=pl.BlockSpec((_EMB, _COLS), lambda i: (4, i)),
        out_shape=jax.ShapeDtypeStruct((_OUT, _BATCH), jnp.float32),
        input_output_aliases={3: 0},
    )(embst, w, b2, outt0)
    return outt


def kernel(ids, feats, adj, layer_idx, table, W, b):
    batch = ids.shape[0]
    gidx = jnp.where(layer_idx > 0, ids.astype(jnp.int32), jnp.int32(_N_NODES))
    gidx = jnp.concatenate([gidx, jnp.zeros((_B_PAD - batch,), jnp.int32)])
    embst = _sc_gather_t(table, gidx)
    outt = _tc_concat_proj(feats, embst, W, b.reshape(_EMB, 1))
    return outt.T


# speedup vs baseline: 3.3660x; 2.2237x over previous
import functools

import jax
import jax.numpy as jnp
from jax import lax
from jax.experimental import pallas as pl
from jax.experimental.pallas import tpu as pltpu
from jax.experimental.pallas import tpu_sc as plsc

_N_NODES = 1000000
_EMB = 32
_FEAT = 128
_OUT = _FEAT + _EMB
_BATCH = 100000

_NC = 2
_NS = 16
_NW = _NC * _NS

_LANES = 128
_TILES_FULL = _N_NODES // _LANES
_TPW = -(-_TILES_FULL // _NW)
_GROUP_T = 16
_GW = _GROUP_T * _LANES
_BOUND_C0 = _TILES_FULL * _LANES
_BOUND_W = 64
_SEL_CAP = 8192
_GRP_CAP = 512
_SCAN_CHUNK = 4000
_OUT_ROWS = _BATCH + _NW
_NG_LAST = (_TILES_FULL - (_NW - 1) * _TPW + _GROUP_T - 1) // _GROUP_T
_NG_MAIN = (_TPW + _GROUP_T - 1) // _GROUP_T


def _sc_gather_native(tablet, idx):
    mesh = plsc.VectorSubcoreMesh(core_axis_name="c", subcore_axis_name="s")

    @functools.partial(
        pl.kernel,
        mesh=mesh,
        out_type=jax.ShapeDtypeStruct((_OUT_ROWS, _LANES), jnp.float32),
        compiler_params=pltpu.CompilerParams(needs_layout_passes=False),
        scratch_types=[
            pltpu.VMEM((_SCAN_CHUNK,), jnp.int32),
            pltpu.VMEM((_SEL_CAP,), jnp.int32),
            pltpu.VMEM((_SEL_CAP,), jnp.int32),
            pltpu.VMEM((_GRP_CAP,), jnp.int32),
            pltpu.VMEM((_GRP_CAP,), jnp.int32),
            pltpu.VMEM((4, 8, _GW), jnp.float32),
            pltpu.VMEM((128, _LANES), jnp.float32),
            pltpu.VMEM((128,), jnp.int32),
            pltpu.SemaphoreType.DMA,
            pltpu.SemaphoreType.DMA,
        ],
    )
    def k(idx_hbm, tab_hbm, out_hbm, sbuf, sel_n, sel_k, grp_n, grp_k,
          tbuf, rows, kstage, fsem, ssem):
        wid = lax.axis_index("s") * _NC + lax.axis_index("c")
        lanes = lax.iota(jnp.int32, 16)
        dump = _BATCH + wid

        lo = wid * (_TPW * _LANES)
        is_last = wid == _NW - 1
        hi = jnp.where(is_last, jnp.int32(2**30), lo + _TPW * _LANES)

        def scan_chunk(c, cnt):
            pltpu.sync_copy(
                idx_hbm.at[pl.ds(c * _SCAN_CHUNK, _SCAN_CHUNK)], sbuf
            )

            def scan_blk(i, cnt):
                iv = sbuf[pl.ds(i * 16, 16)]
                mask = (iv >= lo) & (iv < hi)
                kv = c * _SCAN_CHUNK + i * 16 + lanes
                plsc.store_compressed(sel_n.at[pl.ds(cnt, 16)], iv, mask=mask)
                plsc.store_compressed(sel_k.at[pl.ds(cnt, 16)], kv, mask=mask)
                return cnt + jnp.sum(mask.astype(jnp.int32))

            return lax.fori_loop(0, _SCAN_CHUNK // 16, scan_blk, cnt)

        cnt = lax.fori_loop(0, _BATCH // _SCAN_CHUNK, scan_chunk,
                            jnp.int32(0))
        nsel = (cnt + 15) // 16

        for i in range(8):
            kstage[pl.ds(i * 16, 16)] = jnp.full((16,), dump, jnp.int32)

        def do_group(g_lo, g_hi, c0, width):
            def pre_blk(i, _):
                grp_n[pl.ds(i * 16, 16)] = jnp.full((16,), g_lo, jnp.int32)
                grp_k[pl.ds(i * 16, 16)] = jnp.full((16,), dump, jnp.int32)
                return 0

            lax.fori_loop(0, _GRP_CAP // 16, pre_blk, 0)

            for jt in range(4):
                pltpu.async_copy(
                    tab_hbm.at[pl.ds(jt * 8, 8), pl.ds(c0, width)],
                    tbuf.at[jt].at[:, pl.ds(0, width)],
                    fsem,
                )

            def sel_blk(i, cg):
                nv = sel_n[pl.ds(i * 16, 16)]
                kv = sel_k[pl.ds(i * 16, 16)]
                mask = (nv >= g_lo) & (nv < g_hi)
                plsc.store_compressed(grp_n.at[pl.ds(cg, 16)], nv, mask=mask)
                plsc.store_compressed(grp_k.at[pl.ds(cg, 16)], kv, mask=mask)
                return cg + jnp.sum(mask.astype(jnp.int32))

            cg = lax.fori_loop(0, nsel, sel_blk, jnp.int32(0))

            for jt in range(4):
                pltpu.make_async_copy(
                    tab_hbm.at[pl.ds(jt * 8, 8), pl.ds(c0, width)],
                    tbuf.at[jt].at[:, pl.ds(0, width)],
                    fsem,
                ).wait()

            def ext_blk(b, _):
                nv = grp_n[pl.ds(b * 16, 16)]
                kv = grp_k[pl.ds(b * 16, 16)]
                lv = nv - c0
                r0 = (b % 8) * 16
                kstage[pl.ds(r0, 16)] = kv
                ridx = r0 + lanes
                for j in range(_EMB):
                    vals = plsc.load_gather(
                        tbuf,
                        [jnp.full((16,), j // 8, jnp.int32),
                         jnp.full((16,), j % 8, jnp.int32),
                         lv],
                    )
                    plsc.store_scatter(
                        rows, [ridx, jnp.full((16,), j, jnp.int32)], vals
                    )

                @pl.when(b % 8 == 7)
                def _():
                    pltpu.async_copy(rows, out_hbm.at[kstage], ssem).wait()

                return 0

            nblk = (cg + 15) // 16
            lax.fori_loop(0, nblk, ext_blk, 0)

            pltpu.async_copy(rows, out_hbm.at[kstage], ssem).wait()

        ngroups = jnp.where(is_last, _NG_LAST, _NG_MAIN)

        def group_body(g, _):
            t0 = wid * _TPW + g * _GROUP_T
            t1 = jnp.minimum(t0 + _GROUP_T,
                             jnp.minimum((wid + 1) * _TPW, _TILES_FULL))
            g_lo = t0 * _LANES
            g_hi = t1 * _LANES
            c0 = jnp.minimum(g_lo, (_TILES_FULL - _GROUP_T) * _LANES)
            do_group(g_lo, g_hi, c0, _GW)
            return 0

        lax.fori_loop(0, ngroups, group_body, 0)

        @pl.when(is_last)
        def _():
            do_group(jnp.int32(_BOUND_C0), jnp.int32(_N_NODES),
                     jnp.int32(_BOUND_C0), _BOUND_W)

    return k(idx, tablet)


def _tc_feats_body(feats_ref, out_ref):
    out_ref[...] = feats_ref[...].T


def _tc_emb_body(embs_ref, w_ref, b_ref, last_ref, li_ref, _outp_ref,
                 out_ref):
    e = embs_ref[...][:, :_EMB]
    e2 = jnp.where(
        li_ref[0, 0] > 0, e, jnp.broadcast_to(last_ref[...], e.shape)
    )
    out_ref[...] = (
        jnp.dot(w_ref[...], e2.T, preferred_element_type=jnp.float32)
        + b_ref[...]
    )


_COLS = 2048


def _tc_concat_proj(feats, embs_wide, w, b2, last_row, li):
    grid = pl.cdiv(_BATCH, _COLS)
    outt0 = pl.pallas_call(
        _tc_feats_body,
        grid=(grid,),
        in_specs=[pl.BlockSpec((_COLS, _FEAT), lambda i: (i, 0))],
        out_specs=pl.BlockSpec((_FEAT, _COLS), lambda i: (0, i)),
        out_shape=jax.ShapeDtypeStruct((_OUT, _BATCH), jnp.float32),
    )(feats)
    outt = pl.pallas_call(
        _tc_emb_body,
        grid=(grid,),
        in_specs=[
            pl.BlockSpec((_COLS, _LANES), lambda i: (i, 0)),
            pl.BlockSpec((_EMB, _EMB), lambda i: (0, 0)),
            pl.BlockSpec((_EMB, 1), lambda i: (0, 0)),
            pl.BlockSpec((1, _EMB), lambda i: (0, 0)),
            pl.BlockSpec((1, 1), lambda i: (0, 0)),
            pl.BlockSpec(memory_space=pl.ANY),
        ],
        out_specs=pl.BlockSpec((_EMB, _COLS), lambda i: (4, i)),
        out_shape=jax.ShapeDtypeStruct((_OUT, _BATCH), jnp.float32),
        input_output_aliases={5: 0},
    )(embs_wide, w, b2, last_row, li, outt0)
    return outt


def kernel(ids, feats, adj, layer_idx, table, W, b):
    idx = ids.astype(jnp.int32)
    tablet = jnp.transpose(table)
    embs_wide = _sc_gather_native(tablet, idx)
    last_row = lax.slice(table, (_N_NODES, 0), (_N_NODES + 1, _EMB))
    li = jnp.asarray(layer_idx, jnp.int32).reshape(1, 1)
    outt = _tc_concat_proj(
        feats, embs_wide, W, b.reshape(_EMB, 1), last_row, li
    )
    return outt.T
